# baseline (speedup 1.0000x reference)
import jax
import jax.numpy as jnp
from jax.experimental import pallas as pl


def _dyn_graph_wave_kernel(n1_ref, xp_ref, wself_ref, wagg_ref, out_ref):
    N = n1_ref.shape[0]
    BF = xp_ref.shape[1]
    F = wself_ref.shape[0]
    B = BF // F
    n1 = n1_ref[...]
    logits = jax.lax.dot_general(
        n1, n1, (((1,), (1,)), ((), ())), preferred_element_type=jnp.float32
    )
    s = jax.nn.sigmoid(logits)
    w = jnp.where(s > 0.5, s, 0.0)
    xp = xp_ref[...]
    agg = jax.lax.dot_general(
        w, xp, (((0,), (0,)), ((), ())), preferred_element_type=jnp.float32
    )
    bi = jax.lax.broadcasted_iota(jnp.int32, (BF, BF), 0) // F
    bj = jax.lax.broadcasted_iota(jnp.int32, (BF, BF), 1) // F
    blk = (bi == bj).astype(jnp.float32)
    wagg_blk = jnp.tile(wagg_ref[...], (B, B)) * blk
    wself_blk = jnp.tile(wself_ref[...], (B, B)) * blk
    out_p = (
        jax.lax.dot_general(
            agg, wagg_blk, (((1,), (0,)), ((), ())),
            preferred_element_type=jnp.float32,
        )
        + jax.lax.dot_general(
            xp, wself_blk, (((1,), (0,)), ((), ())),
            preferred_element_type=jnp.float32,
        )
    )
    for b in range(B):
        out_ref[b * N:(b + 1) * N, :] = out_p[:, b * F:(b + 1) * F]


def kernel(x, ptr, node1, W_self, W_agg):
    del ptr
    N, _ = node1.shape
    n, F = x.shape
    B = n // N
    xp = x.reshape(B, N, F).transpose(1, 0, 2).reshape(N, B * F)
    return pl.pallas_call(
        _dyn_graph_wave_kernel,
        out_shape=jax.ShapeDtypeStruct((n, F), x.dtype),
    )(node1, xp, W_self, W_agg)


# speedup vs baseline: 1.2197x; 1.2197x over previous
import jax
import jax.numpy as jnp
from jax.experimental import pallas as pl


def _dyn_graph_wave_kernel(n1_ref, n1t_ref, xp_ref, wself_ref, wagg_ref,
                           out_ref):
    i = pl.program_id(0)
    BF = xp_ref.shape[1]
    F = wself_ref.shape[0]
    B = BF // F
    logits = jax.lax.dot_general(
        n1_ref[...], n1t_ref[...], (((1,), (1,)), ((), ())),
        preferred_element_type=jnp.float32,
    )
    s = jax.nn.sigmoid(logits)
    w = jnp.where(s > 0.5, s, 0.0)
    xp = xp_ref[...]
    agg = jax.lax.dot_general(
        w, xp, (((0,), (0,)), ((), ())), preferred_element_type=jnp.float32
    )
    bi = jax.lax.broadcasted_iota(jnp.int32, (BF, BF), 0) // F
    bj = jax.lax.broadcasted_iota(jnp.int32, (BF, BF), 1) // F
    blk = (bi == bj).astype(jnp.float32)
    wagg_blk = jnp.tile(wagg_ref[...], (B, B)) * blk
    wself_blk = jnp.tile(wself_ref[...], (B, B)) * blk
    CT = out_ref.shape[0]
    xp_tile = xp_ref[pl.ds(i * CT, CT), :]
    out_ref[...] = (
        jax.lax.dot_general(
            agg, wagg_blk, (((1,), (0,)), ((), ())),
            preferred_element_type=jnp.float32,
        )
        + jax.lax.dot_general(
            xp_tile, wself_blk, (((1,), (0,)), ((), ())),
            preferred_element_type=jnp.float32,
        )
    )


def kernel(x, ptr, node1, W_self, W_agg):
    del ptr
    N, D = node1.shape
    n, F = x.shape
    B = n // N
    NT = 2
    CT = N // NT
    xp = x.reshape(B, N, F).transpose(1, 0, 2).reshape(N, B * F)
    out_p = pl.pallas_call(
        _dyn_graph_wave_kernel,
        grid=(NT,),
        in_specs=[
            pl.BlockSpec((N, D), lambda i: (0, 0)),
            pl.BlockSpec((CT, D), lambda i: (i, 0)),
            pl.BlockSpec((N, B * F), lambda i: (0, 0)),
            pl.BlockSpec((F, F), lambda i: (0, 0)),
            pl.BlockSpec((F, F), lambda i: (0, 0)),
        ],
        out_specs=pl.BlockSpec((CT, B * F), lambda i: (i, 0)),
        out_shape=jax.ShapeDtypeStruct((N, B * F), x.dtype),
    )(node1, node1, xp, W_self, W_agg)
    return out_p.reshape(N, B, F).transpose(1, 0, 2).reshape(n, F)


# speedup vs baseline: 1.2782x; 1.0480x over previous
import jax
import jax.numpy as jnp
from jax.experimental import pallas as pl


def _dyn_graph_wave_kernel(n1_ref, xp_ref, wself_ref, wagg_ref, out_ref):
    N = n1_ref.shape[0]
    BF = xp_ref.shape[1]
    F = wself_ref.shape[0]
    B = BF // F
    n1 = n1_ref[...]
    logits = jax.lax.dot_general(
        n1, n1, (((1,), (1,)), ((), ())), preferred_element_type=jnp.float32
    )
    s = jax.nn.sigmoid(logits)
    w = jnp.where(s > 0.5, s, 0.0)
    xp = xp_ref[...]
    agg = jax.lax.dot_general(
        w, xp, (((0,), (0,)), ((), ())), preferred_element_type=jnp.float32
    )
    bi = jax.lax.broadcasted_iota(jnp.int32, (BF, BF), 0) // F
    bj = jax.lax.broadcasted_iota(jnp.int32, (BF, BF), 1) // F
    blk = (bi == bj).astype(jnp.float32)
    wagg_blk = jnp.tile(wagg_ref[...], (B, B)) * blk
    wself_blk = jnp.tile(wself_ref[...], (B, B)) * blk
    out_ref[...] = (
        jax.lax.dot_general(
            agg, wagg_blk, (((1,), (0,)), ((), ())),
            preferred_element_type=jnp.float32,
        )
        + jax.lax.dot_general(
            xp, wself_blk, (((1,), (0,)), ((), ())),
            preferred_element_type=jnp.float32,
        )
    )


def kernel(x, ptr, node1, W_self, W_agg):
    del ptr
    N, _ = node1.shape
    n, F = x.shape
    B = n // N
    xp = x.reshape(B, N, F).transpose(1, 0, 2).reshape(N, B * F)
    out_p = pl.pallas_call(
        _dyn_graph_wave_kernel,
        out_shape=jax.ShapeDtypeStruct((N, B * F), x.dtype),
    )(node1, xp, W_self, W_agg)
    return out_p.reshape(N, B, F).transpose(1, 0, 2).reshape(n, F)


# speedup vs baseline: 1.3400x; 1.0483x over previous
import jax
import jax.numpy as jnp
from jax.experimental import pallas as pl


def _mm(a, b, dims):
    return jax.lax.dot_general(a, b, (dims, ((), ())),
                               preferred_element_type=jnp.float32)


def _dyn_graph_wave_kernel(n1_ref, xp_ref, wself_ref, wagg_ref, out_ref):
    N = n1_ref.shape[0]
    H = N // 2
    BF = xp_ref.shape[1]
    F = wself_ref.shape[0]
    B = BF // F
    n1a = n1_ref[:H, :]
    n1b = n1_ref[H:, :]
    l11 = _mm(n1a, n1a, ((1,), (1,)))
    l12 = _mm(n1a, n1b, ((1,), (1,)))
    l22 = _mm(n1b, n1b, ((1,), (1,)))

    def masked(logits):
        s = jax.nn.sigmoid(logits)
        return jnp.where(s > 0.5, s, 0.0)

    w11 = masked(l11)
    w12 = masked(l12)
    w22 = masked(l22)
    xpa = xp_ref[:H, :]
    xpb = xp_ref[H:, :]
    agg_top = _mm(w11, xpa, ((0,), (0,))) + _mm(w12, xpb, ((1,), (0,)))
    agg_bot = _mm(w12, xpa, ((0,), (0,))) + _mm(w22, xpb, ((0,), (0,)))
    agg = jnp.concatenate([agg_top, agg_bot], axis=0)
    bi = jax.lax.broadcasted_iota(jnp.int32, (BF, BF), 0) // F
    bj = jax.lax.broadcasted_iota(jnp.int32, (BF, BF), 1) // F
    blk = (bi == bj).astype(jnp.float32)
    wagg_blk = jnp.tile(wagg_ref[...], (B, B)) * blk
    wself_blk = jnp.tile(wself_ref[...], (B, B)) * blk
    out_ref[...] = (
        _mm(agg, wagg_blk, ((1,), (0,)))
        + _mm(xp_ref[...], wself_blk, ((1,), (0,)))
    )


def kernel(x, ptr, node1, W_self, W_agg):
    del ptr
    N, _ = node1.shape
    n, F = x.shape
    B = n // N
    xp = x.reshape(B, N, F).transpose(1, 0, 2).reshape(N, B * F)
    out_p = pl.pallas_call(
        _dyn_graph_wave_kernel,
        out_shape=jax.ShapeDtypeStruct((N, B * F), x.dtype),
    )(node1, xp, W_self, W_agg)
    return out_p.reshape(N, B, F).transpose(1, 0, 2).reshape(n, F)


# speedup vs baseline: 1.3809x; 1.0305x over previous
import jax
import jax.numpy as jnp
from jax.experimental import pallas as pl


def _mm(a, b, dims):
    return jax.lax.dot_general(a, b, (dims, ((), ())),
                               preferred_element_type=jnp.float32)


def _dyn_graph_wave_kernel(n1_ref, xp_ref, wself_ref, wagg_ref, out_ref):
    N = n1_ref.shape[0]
    NT = 4
    T = N // NT
    BF = xp_ref.shape[1]
    F = wself_ref.shape[0]
    B = BF // F

    def masked(logits):
        s = jax.nn.sigmoid(logits)
        return jnp.where(s > 0.5, s, 0.0)

    n1t = [n1_ref[i * T:(i + 1) * T, :] for i in range(NT)]
    xpt = [xp_ref[i * T:(i + 1) * T, :] for i in range(NT)]
    w = {}
    for i in range(NT):
        for j in range(i, NT):
            w[(i, j)] = masked(_mm(n1t[i], n1t[j], ((1,), (1,))))
    aggs = []
    for i in range(NT):
        acc = None
        for j in range(NT):
            if j <= i:
                term = _mm(w[(j, i)], xpt[j], ((0,), (0,)))
            else:
                term = _mm(w[(i, j)], xpt[j], ((1,), (0,)))
            acc = term if acc is None else acc + term
        aggs.append(acc)
    agg = jnp.concatenate(aggs, axis=0)
    bi = jax.lax.broadcasted_iota(jnp.int32, (BF, BF), 0) // F
    bj = jax.lax.broadcasted_iota(jnp.int32, (BF, BF), 1) // F
    blk = (bi == bj).astype(jnp.float32)
    wagg_blk = jnp.tile(wagg_ref[...], (B, B)) * blk
    wself_blk = jnp.tile(wself_ref[...], (B, B)) * blk
    out_ref[...] = (
        _mm(agg, wagg_blk, ((1,), (0,)))
        + _mm(xp_ref[...], wself_blk, ((1,), (0,)))
    )


def kernel(x, ptr, node1, W_self, W_agg):
    del ptr
    N, _ = node1.shape
    n, F = x.shape
    B = n // N
    xp = x.reshape(B, N, F).transpose(1, 0, 2).reshape(N, B * F)
    out_p = pl.pallas_call(
        _dyn_graph_wave_kernel,
        out_shape=jax.ShapeDtypeStruct((N, B * F), x.dtype),
    )(node1, xp, W_self, W_agg)
    return out_p.reshape(N, B, F).transpose(1, 0, 2).reshape(n, F)
